# Initial kernel scaffold; baseline (speedup 1.0000x reference)
import jax
import jax.numpy as jnp
from jax.experimental import pallas as pl


def kernel(edge_attr, edge_index, dst_ports, tcp_flags, batch, port_emb, flags_emb, W1_0, b1_0, W2_0, b2_0, W1_1, b1_1, W2_1, b2_1, clf_W1, clf_b1, clf_W2, clf_b2):
    raise NotImplementedError("write your pallas kernel here")


# speedup vs baseline: 1.0867x; 1.0867x over previous
import jax
import jax.numpy as jnp
from jax.experimental import pallas as pl

N = 100000
E = 1600000
NUM_GRAPHS = 64
HIDDEN = 16


def _clf_body(rep_ref, w1_ref, b1_ref, w2_ref, b2_ref, out_ref):
    h = jnp.maximum(rep_ref[...] @ w1_ref[...] + b1_ref[...], 0.0)
    out_ref[...] = h @ w2_ref[...] + b2_ref[...]


def _edge_mpnn(x, src, dst, edge_attr, W1, b1, W2, b2):
    deg = jax.ops.segment_sum(jnp.ones((E,), jnp.float32), dst, num_segments=N)
    loop_attr = jax.ops.segment_sum(edge_attr, dst, num_segments=N) / jnp.maximum(deg, 1.0)[:, None]
    h = jax.nn.relu(edge_attr @ W1 + b1)
    edge_messages = h @ W2 + b2
    agg = jax.ops.segment_sum(jnp.take(x, src, axis=0) + edge_messages, dst, num_segments=N)
    h_loop = jax.nn.relu(loop_attr @ W1 + b1) @ W2 + b2
    return agg + x + h_loop


def kernel(edge_attr, edge_index, dst_ports, tcp_flags, batch, port_emb, flags_emb, W1_0, b1_0, W2_0, b2_0, W1_1, b1_1, W2_1, b2_1, clf_W1, clf_b1, clf_W2, clf_b2):
    dst_emb = jnp.take(port_emb, dst_ports, axis=0)
    fl_emb = jnp.take(flags_emb, tcp_flags, axis=0)
    ea = jnp.concatenate([edge_attr, dst_emb, fl_emb], axis=1)
    src, dst = edge_index[0], edge_index[1]
    x = jnp.zeros((N, 1), jnp.float32)
    x = _edge_mpnn(x, src, dst, ea, W1_0, b1_0, W2_0, b2_0)
    x = _edge_mpnn(x, src, dst, ea, W1_1, b1_1, W2_1, b2_1)
    max_pool = jax.ops.segment_max(x, batch, num_segments=NUM_GRAPHS)
    cnt = jax.ops.segment_sum(jnp.ones((N,), jnp.float32), batch, num_segments=NUM_GRAPHS)
    mean_pool = jax.ops.segment_sum(x, batch, num_segments=NUM_GRAPHS) / jnp.maximum(cnt, 1.0)[:, None]
    graph_rep = jnp.concatenate([max_pool, mean_pool], axis=1)
    return pl.pallas_call(
        _clf_body,
        out_shape=jax.ShapeDtypeStruct((NUM_GRAPHS, clf_W2.shape[1]), jnp.float32),
    )(graph_rep, clf_W1, clf_b1, clf_W2, clf_b2)


# speedup vs baseline: 7.0198x; 6.4595x over previous
import functools

import jax
import jax.numpy as jnp
from jax import lax
from jax.experimental import pallas as pl
from jax.experimental.pallas import tpu as pltpu
from jax.experimental.pallas import tpu_sc as plsc

N = 100000
E = 1600000
H = 16
NUM_GRAPHS = 64
NC, NS = 2, 16
NW = NC * NS
CH = 1024
CPW = 49
TW = CH * CPW
E_PAD = NW * TW
NPAD = 102400
RPT = NPAD // NS
RW = NPAD // NW
PCH = 640
G1 = NUM_GRAPHS + 1

_MESH = plsc.VectorSubcoreMesh(core_axis_name="c", subcore_axis_name="s",
                               num_cores=NC, num_subcores=NS)


def _sc1_body(dst2d, ports, flags, a0, a1, p0, p1, f0, f1, zrows, onerows,
              oz0, or0, oz1, or1, odeg, zs0, zs1,
              acc, idxb, pib, zb, sem):
    c = lax.axis_index("c")
    s = lax.axis_index("s")
    wid = s * NC + c
    ebase = wid * TW
    rbase = wid * (TW // 128)

    def zero_acc():
        pltpu.sync_copy(zrows, acc.at[pl.ds(s * RPT, RPT)])
        plsc.subcore_barrier()

    def flush(out):
        plsc.subcore_barrier()
        pltpu.sync_copy(acc.at[pl.ds(s * RPT, RPT)],
                        out.at[c, pl.ds(s * RPT, RPT)])
        plsc.subcore_barrier()

    def scatter_chunk(i):
        pltpu.sync_copy(dst2d.at[pl.ds(rbase + i * 8, 8)], idxb)
        for j in range(8):
            pltpu.sync_copy(zb.at[pl.ds(j * 128, 128)],
                            acc.at[idxb.at[j]], add=True)

    def phase_z(a, p, f, zstore, out):
        zero_acc()

        @pl.loop(0, CPW)
        def _(i):
            base = ebase + i * CH
            pltpu.sync_copy(a.at[pl.ds(base, CH)], zb)
            pltpu.sync_copy(ports.at[pl.ds(base, CH)], pib)
            pltpu.async_copy(p.at[pib], zb, sem, add=True).wait()
            pltpu.sync_copy(flags.at[pl.ds(base, CH)], pib)
            pltpu.async_copy(f.at[pib], zb, sem, add=True).wait()
            pltpu.sync_copy(zb, zstore.at[pl.ds(base, CH)])
            scatter_chunk(i)

        flush(out)

    def phase_r(zstore, out):
        zero_acc()

        @pl.loop(0, CPW)
        def _(i):
            base = ebase + i * CH
            pltpu.sync_copy(zstore.at[pl.ds(base, CH)], zb)

            @pl.loop(0, CH)
            def _(r):
                zb[r] = jnp.maximum(zb[r], 0.0)

            scatter_chunk(i)

        flush(out)

    def phase_deg(out):
        zero_acc()
        pltpu.sync_copy(onerows, zb)

        @pl.loop(0, CPW)
        def _(i):
            scatter_chunk(i)

        flush(out)

    phase_z(a0, p0, f0, zs0, oz0)
    phase_r(zs0, or0)
    phase_z(a1, p1, f1, zs1, oz1)
    phase_r(zs1, or1)
    phase_deg(odeg)


def _sc2_body(dst2d, srcs, x1, zrows, ogx, acc, idxb, pib, zb, sem):
    c = lax.axis_index("c")
    s = lax.axis_index("s")
    wid = s * NC + c
    ebase = wid * TW
    rbase = wid * (TW // 128)

    pltpu.sync_copy(zrows, acc.at[pl.ds(s * RPT, RPT)])
    plsc.subcore_barrier()

    @pl.loop(0, CPW)
    def _(i):
        base = ebase + i * CH
        pltpu.sync_copy(srcs.at[pl.ds(base, CH)], pib)
        pltpu.async_copy(x1.at[pib], zb, sem).wait()
        pltpu.sync_copy(dst2d.at[pl.ds(rbase + i * 8, 8)], idxb)
        for j in range(8):
            pltpu.sync_copy(zb.at[pl.ds(j * 128, 128)],
                            acc.at[idxb.at[j]], add=True)

    plsc.subcore_barrier()
    pltpu.sync_copy(acc.at[pl.ds(s * RPT, RPT)],
                    ogx.at[c, pl.ds(s * RPT, RPT)])


def _sc3_body(y, gx, batchp, opmax, opsum, opcnt,
              yb, g0b, g1b, bb, amax, asum, acnt, sem):
    c = lax.axis_index("c")
    s = lax.axis_index("s")
    wid = s * NC + c
    base = wid * RW

    @pl.loop(0, G1)
    def _(g):
        amax[g] = jnp.full((H,), -jnp.inf, jnp.float32)
        asum[g] = jnp.zeros((H,), jnp.float32)
        acnt[g] = jnp.zeros((H,), jnp.float32)

    @pl.loop(0, RW // PCH)
    def _(k):
        off = base + k * PCH
        pltpu.sync_copy(y.at[pl.ds(off, PCH)], yb)
        pltpu.sync_copy(gx.at[0, pl.ds(off, PCH)], g0b)
        pltpu.sync_copy(gx.at[1, pl.ds(off, PCH)], g1b)
        pltpu.sync_copy(batchp.at[pl.ds(off, PCH)], bb)

        @pl.loop(0, PCH // 16)
        def _(q):
            gvec = bb[pl.ds(q * 16, 16)]
            for i in range(16):
                r = q * 16 + i
                row = yb[r] + g0b[r] + g1b[r]
                g = gvec[i]
                amax[g] = jnp.maximum(amax[g], row)
                asum[g] = asum[g] + row
                acnt[g] = acnt[g] + 1.0

    pltpu.sync_copy(amax, opmax.at[wid])
    pltpu.sync_copy(asum, opsum.at[wid])
    pltpu.sync_copy(acnt, opcnt.at[wid])


def _mm_body(x_ref, w_ref, b_ref, o_ref):
    o_ref[...] = jnp.dot(x_ref[...], w_ref[...],
                         preferred_element_type=jnp.float32) + b_ref[...]


def _mm(x, w, b, bm):
    m, k = x.shape
    n = w.shape[1]
    return pl.pallas_call(
        _mm_body,
        grid=(m // bm,),
        in_specs=[pl.BlockSpec((bm, k), lambda i: (i, 0)),
                  pl.BlockSpec((k, n), lambda i: (0, 0)),
                  pl.BlockSpec((1, n), lambda i: (0, 0))],
        out_specs=pl.BlockSpec((bm, n), lambda i: (i, 0)),
        out_shape=jax.ShapeDtypeStruct((m, n), jnp.float32),
    )(x, w, b.reshape(1, -1))


def _t2_body(z0p, s0p, degp, w2, b1, b2, x1o):
    w2v, b1v, b2v = w2[...], b1[...], b2[...]
    z0 = z0p[0] + z0p[1]
    s0 = s0p[0] + s0p[1]
    deg = degp[0] + degp[1]
    degc = jnp.maximum(deg, 1.0)
    l0 = jnp.maximum((z0 - deg * b1v) / degc + b1v, 0.0)
    x1o[...] = jnp.dot(s0 + l0, w2v,
                       preferred_element_type=jnp.float32) + (deg + 1.0) * b2v


def _t3_body(x1, z1p, s1p, degp, w2, b1, b2, yo):
    w2v, b1v, b2v = w2[...], b1[...], b2[...]
    z1 = z1p[0] + z1p[1]
    s1 = s1p[0] + s1p[1]
    deg = degp[0] + degp[1]
    degc = jnp.maximum(deg, 1.0)
    l1 = jnp.maximum((z1 - deg * b1v) / degc + b1v, 0.0)
    yo[...] = x1[...] + jnp.dot(s1 + l1, w2v,
                                preferred_element_type=jnp.float32) \
        + (deg + 1.0) * b2v


def _t4_body(pmax, psum, pcnt, w1, b1, w2, b2, o_ref):
    mx = jnp.max(pmax[...][:, :NUM_GRAPHS, :], axis=0)
    sm = jnp.sum(psum[...][:, :NUM_GRAPHS, :], axis=0)
    cnt = jnp.sum(pcnt[...][:, :NUM_GRAPHS, :1], axis=0)
    mean = sm / jnp.maximum(cnt, 1.0)
    rep = jnp.concatenate([mx, mean], axis=1)
    h = jnp.maximum(jnp.dot(rep, w1[...], preferred_element_type=jnp.float32)
                    + b1[...], 0.0)
    o_ref[...] = jnp.dot(h, w2[...], preferred_element_type=jnp.float32) \
        + b2[...]


def kernel(edge_attr, edge_index, dst_ports, tcp_flags, batch, port_emb,
           flags_emb, W1_0, b1_0, W2_0, b2_0, W1_1, b1_1, W2_1, b2_1,
           clf_W1, clf_b1, clf_W2, clf_b2):
    pad = E_PAD - E
    src = edge_index[0]
    dst = edge_index[1]
    dstp = jnp.concatenate([dst, jnp.full((pad,), N, jnp.int32)])
    dst2d = dstp.reshape(-1, 128)
    srcp = jnp.concatenate([src, jnp.zeros((pad,), jnp.int32)])
    portsp = jnp.concatenate([dst_ports, jnp.zeros((pad,), jnp.int32)])
    flagsp = jnp.concatenate([tcp_flags, jnp.zeros((pad,), jnp.int32)])
    eap = jnp.concatenate([edge_attr, jnp.zeros((pad, H), jnp.float32)])
    batchp = jnp.concatenate(
        [batch, jnp.full((NPAD - N,), NUM_GRAPHS, jnp.int32)])
    zeros_h = jnp.zeros((H,), jnp.float32)
    zrows = jnp.zeros((RPT, H), jnp.float32)
    onerows = jnp.ones((CH, H), jnp.float32)

    A0 = _mm(eap, W1_0[:H], zeros_h, 8192)
    A1 = _mm(eap, W1_1[:H], zeros_h, 8192)
    P0 = _mm(port_emb, W1_0[H:2 * H], zeros_h, 8192)
    P1 = _mm(port_emb, W1_1[H:2 * H], zeros_h, 8192)
    F0 = _mm(flags_emb, W1_0[2 * H:], b1_0, 256)
    F1 = _mm(flags_emb, W1_1[2 * H:], b1_1, 256)

    sc1 = pl.kernel(
        _sc1_body,
        out_type=[jax.ShapeDtypeStruct((NC, NPAD, H), jnp.float32)] * 5
        + [jax.ShapeDtypeStruct((E_PAD, H), jnp.float32)] * 2,
        mesh=_MESH,
        compiler_params=pltpu.CompilerParams(use_tc_tiling_on_sc=False),
        scratch_types=[
            pltpu.VMEM_SHARED((NPAD, H), jnp.float32),
            pltpu.VMEM((8, 128), jnp.int32),
            pltpu.VMEM((CH,), jnp.int32),
            pltpu.VMEM((CH, H), jnp.float32),
            pltpu.SemaphoreType.DMA,
        ],
    )
    oz0, or0, oz1, or1, odeg, _, _ = sc1(
        dst2d, portsp, flagsp, A0, A1, P0, P1, F0, F1, zrows, onerows)

    bn = 3200
    node_specs = [pl.BlockSpec((NC, bn, H), lambda i: (0, i, 0))] * 3
    w_specs = [pl.BlockSpec((H, H), lambda i: (0, 0)),
               pl.BlockSpec((1, H), lambda i: (0, 0)),
               pl.BlockSpec((1, H), lambda i: (0, 0))]
    x1 = pl.pallas_call(
        _t2_body,
        grid=(NPAD // bn,),
        in_specs=node_specs + w_specs,
        out_specs=pl.BlockSpec((bn, H), lambda i: (i, 0)),
        out_shape=jax.ShapeDtypeStruct((NPAD, H), jnp.float32),
    )(oz0, or0, odeg, W2_0, b1_0.reshape(1, -1), b2_0.reshape(1, -1))

    sc2 = pl.kernel(
        _sc2_body,
        out_type=jax.ShapeDtypeStruct((NC, NPAD, H), jnp.float32),
        mesh=_MESH,
        compiler_params=pltpu.CompilerParams(use_tc_tiling_on_sc=False),
        scratch_types=[
            pltpu.VMEM_SHARED((NPAD, H), jnp.float32),
            pltpu.VMEM((8, 128), jnp.int32),
            pltpu.VMEM((CH,), jnp.int32),
            pltpu.VMEM((CH, H), jnp.float32),
            pltpu.SemaphoreType.DMA,
        ],
    )
    ogx = sc2(dst2d, srcp, x1, zrows)

    y = pl.pallas_call(
        _t3_body,
        grid=(NPAD // bn,),
        in_specs=[pl.BlockSpec((bn, H), lambda i: (i, 0))]
        + node_specs + w_specs,
        out_specs=pl.BlockSpec((bn, H), lambda i: (i, 0)),
        out_shape=jax.ShapeDtypeStruct((NPAD, H), jnp.float32),
    )(x1, oz1, or1, odeg, W2_1, b1_1.reshape(1, -1), b2_1.reshape(1, -1))

    sc3 = pl.kernel(
        _sc3_body,
        out_type=[jax.ShapeDtypeStruct((NW, G1, H), jnp.float32)] * 3,
        mesh=_MESH,
        compiler_params=pltpu.CompilerParams(use_tc_tiling_on_sc=False),
        scratch_types=[
            pltpu.VMEM((PCH, H), jnp.float32),
            pltpu.VMEM((PCH, H), jnp.float32),
            pltpu.VMEM((PCH, H), jnp.float32),
            pltpu.VMEM((PCH,), jnp.int32),
            pltpu.VMEM((G1, H), jnp.float32),
            pltpu.VMEM((G1, H), jnp.float32),
            pltpu.VMEM((G1, H), jnp.float32),
            pltpu.SemaphoreType.DMA,
        ],
    )
    pmax, psum, pcnt = sc3(y, ogx, batchp)

    return pl.pallas_call(
        _t4_body,
        out_shape=jax.ShapeDtypeStruct((NUM_GRAPHS, clf_W2.shape[1]),
                                       jnp.float32),
    )(pmax, psum, pcnt, clf_W1, clf_b1.reshape(1, -1), clf_W2,
      clf_b2.reshape(1, -1))


# speedup vs baseline: 7.3341x; 1.0448x over previous
import functools

import jax
import jax.numpy as jnp
from jax import lax
from jax.experimental import pallas as pl
from jax.experimental.pallas import tpu as pltpu
from jax.experimental.pallas import tpu_sc as plsc

N = 100000
E = 1600000
H = 16
NUM_GRAPHS = 64
NC, NS = 2, 16
NW = NC * NS
CH = 512
CPW = 98
TW = CH * CPW
E_PAD = NW * TW
NPAD = 102400
RPT = NPAD // NS
RW = NPAD // NW
PCH = 640
G1 = NUM_GRAPHS + 1

_MESH = plsc.VectorSubcoreMesh(core_axis_name="c", subcore_axis_name="s",
                               num_cores=NC, num_subcores=NS)


def _sc1_body(dst2d, ports, flags, a0, a1, p0, p1, f0, f1, zrows, onerows,
              oz0, or0, oz1, or1, odeg,
              acc, idxb, pib, pg, ab, sem):
    c = lax.axis_index("c")
    s = lax.axis_index("s")
    wid = s * NC + c
    ebase = wid * TW
    rbase = wid * (TW // 128)

    def zero_acc():
        pltpu.sync_copy(zrows, acc.at[pl.ds(s * RPT, RPT)])
        plsc.subcore_barrier()

    def flush(out):
        plsc.subcore_barrier()
        pltpu.sync_copy(acc.at[pl.ds(s * RPT, RPT)],
                        out.at[c, pl.ds(s * RPT, RPT)])
        plsc.subcore_barrier()

    def scatter_chunk(i):
        pltpu.sync_copy(dst2d.at[pl.ds(rbase + i * (CH // 128), CH // 128)],
                        idxb)
        for j in range(CH // 128):
            pltpu.sync_copy(pg.at[pl.ds(j * 128, 128)],
                            acc.at[idxb.at[j]], add=True)

    def phase_zr(a, p, f, out, do_relu):
        zero_acc()

        @pl.loop(0, CPW)
        def _(i):
            base = ebase + i * CH
            pltpu.sync_copy(ports.at[pl.ds(base, CH)], pib)
            pltpu.async_copy(p.at[pib], pg, sem).wait()
            pltpu.sync_copy(flags.at[pl.ds(base, CH)], pib)
            pltpu.async_copy(f.at[pib], pg, sem, add=True).wait()
            pltpu.sync_copy(a.at[pl.ds(base // 8, CH // 8)], ab)

            @pl.loop(0, CH // 8, unroll=2)
            def _(q):
                for k in range(8):
                    r = q * 8 + k
                    v = pg[r] + ab[q, pl.ds(k * H, H)]
                    if do_relu:
                        v = jnp.maximum(v, 0.0)
                    pg[r] = v

            scatter_chunk(i)

        flush(out)

    def phase_deg(out):
        zero_acc()
        pltpu.sync_copy(onerows, pg)

        @pl.loop(0, CPW)
        def _(i):
            scatter_chunk(i)

        flush(out)

    phase_zr(a0, p0, f0, oz0, False)
    phase_zr(a0, p0, f0, or0, True)
    phase_zr(a1, p1, f1, oz1, False)
    phase_zr(a1, p1, f1, or1, True)
    phase_deg(odeg)


def _sc2_body(dst2d, srcs, x1, zrows, ogx, acc, idxb, pib, zb, sem):
    c = lax.axis_index("c")
    s = lax.axis_index("s")
    wid = s * NC + c
    ebase = wid * TW
    rbase = wid * (TW // 128)

    pltpu.sync_copy(zrows, acc.at[pl.ds(s * RPT, RPT)])
    plsc.subcore_barrier()

    @pl.loop(0, CPW)
    def _(i):
        base = ebase + i * CH
        pltpu.sync_copy(srcs.at[pl.ds(base, CH)], pib)
        pltpu.async_copy(x1.at[pib], zb, sem).wait()
        pltpu.sync_copy(dst2d.at[pl.ds(rbase + i * (CH // 128), CH // 128)],
                        idxb)
        for j in range(CH // 128):
            pltpu.sync_copy(zb.at[pl.ds(j * 128, 128)],
                            acc.at[idxb.at[j]], add=True)

    plsc.subcore_barrier()
    pltpu.sync_copy(acc.at[pl.ds(s * RPT, RPT)],
                    ogx.at[c, pl.ds(s * RPT, RPT)])


def _sc3_body(y, gx, batchp, opmax, opsum, opcnt,
              yb, g0b, g1b, bb, amax, asum, acnt, sem):
    c = lax.axis_index("c")
    s = lax.axis_index("s")
    wid = s * NC + c
    base = wid * RW

    @pl.loop(0, G1)
    def _(g):
        amax[g] = jnp.full((H,), -jnp.inf, jnp.float32)
        asum[g] = jnp.zeros((H,), jnp.float32)
        acnt[g] = jnp.zeros((H,), jnp.float32)

    @pl.loop(0, RW // PCH)
    def _(k):
        off = base + k * PCH
        pltpu.sync_copy(y.at[pl.ds(off, PCH)], yb)
        pltpu.sync_copy(gx.at[0, pl.ds(off, PCH)], g0b)
        pltpu.sync_copy(gx.at[1, pl.ds(off, PCH)], g1b)
        pltpu.sync_copy(batchp.at[pl.ds(off, PCH)], bb)

        @pl.loop(0, PCH // 16)
        def _(q):
            gvec = bb[pl.ds(q * 16, 16)]
            for i in range(16):
                r = q * 16 + i
                row = yb[r] + g0b[r] + g1b[r]
                g = gvec[i]
                amax[g] = jnp.maximum(amax[g], row)
                asum[g] = asum[g] + row
                acnt[g] = acnt[g] + 1.0

    pltpu.sync_copy(amax, opmax.at[wid])
    pltpu.sync_copy(asum, opsum.at[wid])
    pltpu.sync_copy(acnt, opcnt.at[wid])


def _mm_body(x_ref, w_ref, b_ref, o_ref):
    o_ref[...] = jnp.dot(x_ref[...], w_ref[...],
                         preferred_element_type=jnp.float32) + b_ref[...]


def _mm(x, w, b, bm):
    m, k = x.shape
    n = w.shape[1]
    return pl.pallas_call(
        _mm_body,
        grid=(m // bm,),
        in_specs=[pl.BlockSpec((bm, k), lambda i: (i, 0)),
                  pl.BlockSpec((k, n), lambda i: (0, 0)),
                  pl.BlockSpec((1, n), lambda i: (0, 0))],
        out_specs=pl.BlockSpec((bm, n), lambda i: (i, 0)),
        out_shape=jax.ShapeDtypeStruct((m, n), jnp.float32),
    )(x, w, b.reshape(1, -1))


def _mm2_body(x_ref, w0_ref, w1_ref, o0_ref, o1_ref):
    x = x_ref[...]
    o0_ref[...] = jnp.dot(x, w0_ref[...], preferred_element_type=jnp.float32)
    o1_ref[...] = jnp.dot(x, w1_ref[...], preferred_element_type=jnp.float32)


def _mm2(x, w0, w1, bm):
    m, k = x.shape
    out = jax.ShapeDtypeStruct((m, k), jnp.float32)
    return pl.pallas_call(
        _mm2_body,
        grid=(m // bm,),
        in_specs=[pl.BlockSpec((bm, k), lambda i: (i, 0)),
                  pl.BlockSpec((k, k), lambda i: (0, 0)),
                  pl.BlockSpec((k, k), lambda i: (0, 0))],
        out_specs=[pl.BlockSpec((bm, k), lambda i: (i, 0))] * 2,
        out_shape=[out, out],
    )(x, w0, w1)


def _t2_body(z0p, s0p, degp, w2, b1, b2, x1o):
    w2v, b1v, b2v = w2[...], b1[...], b2[...]
    z0 = z0p[0] + z0p[1]
    s0 = s0p[0] + s0p[1]
    deg = degp[0] + degp[1]
    degc = jnp.maximum(deg, 1.0)
    l0 = jnp.maximum((z0 - deg * b1v) / degc + b1v, 0.0)
    x1o[...] = jnp.dot(s0 + l0, w2v,
                       preferred_element_type=jnp.float32) + (deg + 1.0) * b2v


def _t3_body(x1, z1p, s1p, degp, w2, b1, b2, yo):
    w2v, b1v, b2v = w2[...], b1[...], b2[...]
    z1 = z1p[0] + z1p[1]
    s1 = s1p[0] + s1p[1]
    deg = degp[0] + degp[1]
    degc = jnp.maximum(deg, 1.0)
    l1 = jnp.maximum((z1 - deg * b1v) / degc + b1v, 0.0)
    yo[...] = x1[...] + jnp.dot(s1 + l1, w2v,
                                preferred_element_type=jnp.float32) \
        + (deg + 1.0) * b2v


def _t4_body(pmax, psum, pcnt, w1, b1, w2, b2, o_ref):
    mx = jnp.max(pmax[...][:, :NUM_GRAPHS, :], axis=0)
    sm = jnp.sum(psum[...][:, :NUM_GRAPHS, :], axis=0)
    cnt = jnp.sum(pcnt[...][:, :NUM_GRAPHS, :1], axis=0)
    mean = sm / jnp.maximum(cnt, 1.0)
    rep = jnp.concatenate([mx, mean], axis=1)
    h = jnp.maximum(jnp.dot(rep, w1[...], preferred_element_type=jnp.float32)
                    + b1[...], 0.0)
    o_ref[...] = jnp.dot(h, w2[...], preferred_element_type=jnp.float32) \
        + b2[...]


def kernel(edge_attr, edge_index, dst_ports, tcp_flags, batch, port_emb,
           flags_emb, W1_0, b1_0, W2_0, b2_0, W1_1, b1_1, W2_1, b2_1,
           clf_W1, clf_b1, clf_W2, clf_b2):
    pad = E_PAD - E
    src = edge_index[0]
    dst = edge_index[1]
    dstp = jnp.concatenate([dst, jnp.full((pad,), N, jnp.int32)])
    dst2d = dstp.reshape(-1, 128)
    srcp = jnp.concatenate([src, jnp.zeros((pad,), jnp.int32)])
    portsp = jnp.concatenate([dst_ports, jnp.zeros((pad,), jnp.int32)])
    flagsp = jnp.concatenate([tcp_flags, jnp.zeros((pad,), jnp.int32)])
    batchp = jnp.concatenate(
        [batch, jnp.full((NPAD - N,), NUM_GRAPHS, jnp.int32)])
    zeros_h = jnp.zeros((H,), jnp.float32)
    zrows = jnp.zeros((RPT, H), jnp.float32)
    onerows = jnp.ones((CH, H), jnp.float32)

    eye8 = jnp.eye(8, dtype=jnp.float32)
    BW0 = jnp.kron(eye8, W1_0[:H])
    BW1 = jnp.kron(eye8, W1_1[:H])
    ea128 = jnp.pad(edge_attr.reshape(E * H // 128, 128),
                    ((0, (E_PAD - E) * H // 128), (0, 0)))
    A0, A1 = _mm2(ea128, BW0, BW1, 2048)
    P0 = _mm(port_emb, W1_0[H:2 * H], zeros_h, 8192)
    P1 = _mm(port_emb, W1_1[H:2 * H], zeros_h, 8192)
    F0 = _mm(flags_emb, W1_0[2 * H:], b1_0, 256)
    F1 = _mm(flags_emb, W1_1[2 * H:], b1_1, 256)

    sc1 = pl.kernel(
        _sc1_body,
        out_type=[jax.ShapeDtypeStruct((NC, NPAD, H), jnp.float32)] * 5,
        mesh=_MESH,
        compiler_params=pltpu.CompilerParams(use_tc_tiling_on_sc=False),
        scratch_types=[
            pltpu.VMEM_SHARED((NPAD, H), jnp.float32),
            pltpu.VMEM((CH // 128, 128), jnp.int32),
            pltpu.VMEM((CH,), jnp.int32),
            pltpu.VMEM((CH, H), jnp.float32),
            pltpu.VMEM((CH // 8, 128), jnp.float32),
            pltpu.SemaphoreType.DMA,
        ],
    )
    oz0, or0, oz1, or1, odeg = sc1(
        dst2d, portsp, flagsp, A0, A1, P0, P1, F0, F1, zrows, onerows)

    bn = 3200
    node_specs = [pl.BlockSpec((NC, bn, H), lambda i: (0, i, 0))] * 3
    w_specs = [pl.BlockSpec((H, H), lambda i: (0, 0)),
               pl.BlockSpec((1, H), lambda i: (0, 0)),
               pl.BlockSpec((1, H), lambda i: (0, 0))]
    x1 = pl.pallas_call(
        _t2_body,
        grid=(NPAD // bn,),
        in_specs=node_specs + w_specs,
        out_specs=pl.BlockSpec((bn, H), lambda i: (i, 0)),
        out_shape=jax.ShapeDtypeStruct((NPAD, H), jnp.float32),
    )(oz0, or0, odeg, W2_0, b1_0.reshape(1, -1), b2_0.reshape(1, -1))

    sc2 = pl.kernel(
        _sc2_body,
        out_type=jax.ShapeDtypeStruct((NC, NPAD, H), jnp.float32),
        mesh=_MESH,
        compiler_params=pltpu.CompilerParams(use_tc_tiling_on_sc=False),
        scratch_types=[
            pltpu.VMEM_SHARED((NPAD, H), jnp.float32),
            pltpu.VMEM((CH // 128, 128), jnp.int32),
            pltpu.VMEM((CH,), jnp.int32),
            pltpu.VMEM((CH, H), jnp.float32),
            pltpu.SemaphoreType.DMA,
        ],
    )
    ogx = sc2(dst2d, srcp, x1, zrows)

    y = pl.pallas_call(
        _t3_body,
        grid=(NPAD // bn,),
        in_specs=[pl.BlockSpec((bn, H), lambda i: (i, 0))]
        + node_specs + w_specs,
        out_specs=pl.BlockSpec((bn, H), lambda i: (i, 0)),
        out_shape=jax.ShapeDtypeStruct((NPAD, H), jnp.float32),
    )(x1, oz1, or1, odeg, W2_1, b1_1.reshape(1, -1), b2_1.reshape(1, -1))

    sc3 = pl.kernel(
        _sc3_body,
        out_type=[jax.ShapeDtypeStruct((NW, G1, H), jnp.float32)] * 3,
        mesh=_MESH,
        compiler_params=pltpu.CompilerParams(use_tc_tiling_on_sc=False),
        scratch_types=[
            pltpu.VMEM((PCH, H), jnp.float32),
            pltpu.VMEM((PCH, H), jnp.float32),
            pltpu.VMEM((PCH, H), jnp.float32),
            pltpu.VMEM((PCH,), jnp.int32),
            pltpu.VMEM((G1, H), jnp.float32),
            pltpu.VMEM((G1, H), jnp.float32),
            pltpu.VMEM((G1, H), jnp.float32),
            pltpu.SemaphoreType.DMA,
        ],
    )
    pmax, psum, pcnt = sc3(y, ogx, batchp)

    return pl.pallas_call(
        _t4_body,
        out_shape=jax.ShapeDtypeStruct((NUM_GRAPHS, clf_W2.shape[1]),
                                       jnp.float32),
    )(pmax, psum, pcnt, clf_W1, clf_b1.reshape(1, -1), clf_W2,
      clf_b2.reshape(1, -1))
